# baseline (speedup 1.0000x reference)
import functools

import jax
import jax.numpy as jnp
from jax import lax
from jax.experimental import pallas as pl
from jax.experimental.pallas import tpu as pltpu
from jax.experimental.pallas import tpu_sc as plsc


def _make_sc_kernel(B, D, NC, NS, L):
    NW = NC * NS
    b_per_w = B // NW
    CH = 128
    n_ch = b_per_w // CH
    H = D // 2

    mesh = plsc.VectorSubcoreMesh(
        core_axis_name="c", subcore_axis_name="s",
        num_cores=NC, num_subcores=NS)

    @functools.partial(
        pl.kernel,
        mesh=mesh,
        compiler_params=pltpu.CompilerParams(
            use_tc_tiling_on_sc=False, needs_layout_passes=False),
        out_type=jax.ShapeDtypeStruct((NW, L), jnp.float32),
        scratch_types=[
            pltpu.VMEM((n_ch, CH), jnp.int32),
            pltpu.VMEM((b_per_w, H), jnp.float32),
            pltpu.VMEM((b_per_w, H), jnp.float32),
            pltpu.VMEM((b_per_w, D), jnp.float32),
            pltpu.VMEM((L,), jnp.float32),
            pltpu.SemaphoreType.DMA,
        ],
    )
    def sc_kernel(tab_a_hbm, tab_b_hbm, feat_hbm, lab_hbm, out_hbm,
                  idx_v, rows_a, rows_b, feat_v, acc_v, sem):
        wid = lax.axis_index("s") * NC + lax.axis_index("c")
        pltpu.sync_copy(lab_hbm.at[wid], idx_v)
        copies = [
            pltpu.make_async_copy(
                tab.at[idx_v.at[j]],
                rows.at[pl.ds(j * CH, CH), :],
                sem)
            for tab, rows in ((tab_a_hbm, rows_a), (tab_b_hbm, rows_b))
            for j in range(n_ch)
        ]
        for cp in copies:
            cp.start()
        pltpu.sync_copy(feat_hbm.at[wid], feat_v)
        for cp in copies:
            cp.wait()

        def body(i, acc):
            a0 = rows_a[i, :] * feat_v[i, pl.ds(0, H)]
            a1 = rows_b[i, :] * feat_v[i, pl.ds(H, H)]
            return acc + a0 + a1

        acc = lax.fori_loop(0, b_per_w, body,
                            jnp.zeros((L,), jnp.float32))
        acc_v[...] = acc
        pltpu.sync_copy(acc_v, out_hbm.at[wid])

    return sc_kernel


def kernel(centres, features, labels):
    B, D = features.shape
    info = plsc.get_sparse_core_info()
    NC, NS, L = info.num_cores, info.num_subcores, info.num_lanes
    NW = NC * NS
    b_per_w = B // NW
    lab = labels.astype(jnp.int32).reshape(NW, b_per_w // 128, 128)
    feat = features.reshape(NW, b_per_w, D)
    tab_a = centres[:, : D // 2]
    tab_b = centres[:, D // 2:]
    partials = _make_sc_kernel(B, D, NC, NS, L)(tab_a, tab_b, feat, lab)
    return jnp.sum(partials)


# speedup vs baseline: 2.3581x; 2.3581x over previous
import functools

import jax
import jax.numpy as jnp
from jax import lax
from jax.experimental import pallas as pl
from jax.experimental.pallas import tpu as pltpu
from jax.experimental.pallas import tpu_sc as plsc


def _make_sc_kernel(B, D, NC, NS, L):
    NW = NC * NS
    b_per_w = B // NW
    CH = 128
    n_ch = b_per_w // CH
    n_g = b_per_w // L

    mesh = plsc.VectorSubcoreMesh(
        core_axis_name="c", subcore_axis_name="s",
        num_cores=NC, num_subcores=NS)

    @functools.partial(
        pl.kernel,
        mesh=mesh,
        compiler_params=pltpu.CompilerParams(needs_layout_passes=False),
        out_type=jax.ShapeDtypeStruct((NW, L), jnp.float32),
        scratch_types=[
            pltpu.VMEM((n_ch, CH), jnp.int32),
            pltpu.VMEM((n_ch, CH), jnp.int32),
            pltpu.VMEM((b_per_w, 4 * D), jnp.float32),
            pltpu.VMEM((D, b_per_w), jnp.float32),
            pltpu.VMEM((L,), jnp.float32),
            pltpu.SemaphoreType.DMA,
        ],
    )
    def sc_kernel(table_hbm, feat_t_hbm, line_hbm, lab_hbm, out_hbm,
                  line_v, lab_v, rows_v, feat_v, acc_v, sem):
        wid = lax.axis_index("s") * NC + lax.axis_index("c")
        pltpu.sync_copy(line_hbm.at[wid], line_v)
        pltpu.sync_copy(lab_hbm.at[wid], lab_v)
        copies = [
            pltpu.make_async_copy(
                table_hbm.at[line_v.at[j]],
                rows_v.at[pl.ds(j * CH, CH), :],
                sem)
            for j in range(n_ch)
        ]
        for cp in copies:
            cp.start()
        pltpu.sync_copy(feat_t_hbm.at[:, pl.ds(wid * b_per_w, b_per_w)],
                        feat_v)
        for cp in copies:
            cp.wait()

        lane = lax.iota(jnp.int32, L)
        acc = jnp.zeros((L,), jnp.float32)
        for k in range(n_g):
            l_vec = k * L + lane
            labs = lab_v[k // (CH // L), pl.ds((k % (CH // L)) * L, L)]
            o_vec = (labs & 3) * D

            def body(j, a, l_vec=l_vec, o_vec=o_vec, k=k):
                g = plsc.load_gather(rows_v, [l_vec, o_vec + j])
                return a + g * feat_v[j, pl.ds(k * L, L)]

            acc = lax.fori_loop(0, D, body, acc)
        acc_v[...] = acc
        pltpu.sync_copy(acc_v, out_hbm.at[wid])

    return sc_kernel


def kernel(centres, features, labels):
    B, D = features.shape
    V = centres.shape[0]
    info = plsc.get_sparse_core_info()
    NC, NS, L = info.num_cores, info.num_subcores, info.num_lanes
    NW = NC * NS
    b_per_w = B // NW
    lab32 = labels.astype(jnp.int32)
    table = centres.reshape(V // 4, 4 * D)
    lines = (lab32 // 4).reshape(NW, b_per_w // 128, 128)
    lab3d = lab32.reshape(NW, b_per_w // 128, 128)
    feat_t = features.T
    partials = _make_sc_kernel(B, D, NC, NS, L)(table, feat_t, lines, lab3d)
    return jnp.sum(partials)


# speedup vs baseline: 2.3911x; 1.0140x over previous
import functools

import jax
import jax.numpy as jnp
from jax import lax
from jax.experimental import pallas as pl
from jax.experimental.pallas import tpu as pltpu
from jax.experimental.pallas import tpu_sc as plsc


def _make_sc_kernel(B, D, NC, NS, L):
    NW = NC * NS
    b_per_w = B // NW
    CH = 128
    n_ch = b_per_w // CH

    mesh = plsc.VectorSubcoreMesh(
        core_axis_name="c", subcore_axis_name="s",
        num_cores=NC, num_subcores=NS)

    @functools.partial(
        pl.kernel,
        mesh=mesh,
        compiler_params=pltpu.CompilerParams(use_tc_tiling_on_sc=False),
        out_type=jax.ShapeDtypeStruct((NW, L), jnp.float32),
        scratch_types=[
            pltpu.VMEM((n_ch, CH), jnp.int32),
            pltpu.VMEM((b_per_w, D), jnp.float32),
            pltpu.VMEM((b_per_w, D), jnp.float32),
            pltpu.VMEM((L,), jnp.float32),
            pltpu.SemaphoreType.DMA,
        ],
    )
    def sc_kernel(centres_hbm, feat_hbm, lab_hbm, out_hbm,
                  idx_v, rows_v, feat_v, acc_v, sem):
        wid = lax.axis_index("s") * NC + lax.axis_index("c")
        pltpu.sync_copy(lab_hbm.at[wid], idx_v)
        copies = [
            pltpu.make_async_copy(
                centres_hbm.at[idx_v.at[j]],
                rows_v.at[pl.ds(j * CH, CH)],
                sem)
            for j in range(n_ch)
        ]
        for c in copies:
            c.start()
        pltpu.sync_copy(feat_hbm.at[wid], feat_v)
        for c in copies:
            c.wait()

        def body(i, acc):
            a0 = rows_v[i, pl.ds(0, L)] * feat_v[i, pl.ds(0, L)]
            a1 = rows_v[i, pl.ds(L, L)] * feat_v[i, pl.ds(L, L)]
            return acc + a0 + a1

        acc = lax.fori_loop(0, b_per_w, body,
                            jnp.zeros((L,), jnp.float32))
        acc_v[...] = acc
        pltpu.sync_copy(acc_v, out_hbm.at[wid])

    return sc_kernel


def kernel(centres, features, labels):
    B, D = features.shape
    info = plsc.get_sparse_core_info()
    NC, NS, L = info.num_cores, info.num_subcores, info.num_lanes
    NW = NC * NS
    b_per_w = B // NW
    lab = labels.astype(jnp.int32).reshape(NW, b_per_w // 128, 128)
    feat = features.reshape(NW, b_per_w, D)
    partials = _make_sc_kernel(B, D, NC, NS, L)(centres, feat, lab)
    return jnp.sum(partials)
